# baseline (speedup 1.0000x reference)
import functools

import jax
import jax.numpy as jnp
from jax.experimental import pallas as pl
from jax.experimental.pallas import tpu as pltpu

F32 = jnp.float32
CDT = jnp.bfloat16


def _mm_body(a_ref, b_ref, o_ref, acc_ref, *, nk, trans_a):
    k = pl.program_id(2)

    @pl.when(k == 0)
    def _():
        acc_ref[...] = jnp.zeros_like(acc_ref)

    a = a_ref[...]
    b = b_ref[...]
    if trans_a:
        acc_ref[...] += jax.lax.dot_general(
            a, b, (((0,), (0,)), ((), ())), preferred_element_type=F32)
    else:
        acc_ref[...] += jnp.dot(a, b, preferred_element_type=F32)

    @pl.when(k == nk - 1)
    def _():
        o_ref[...] = acc_ref[...].astype(o_ref.dtype)


def _pick(dim, cands):
    for c in cands:
        if dim % c == 0:
            return c
    return dim


def _mm(a, b, *, trans_a=False, out_dtype=None, bm=None, bk=None):
    if trans_a:
        kdim, m = a.shape
    else:
        m, kdim = a.shape
    _, n = b.shape
    bm = bm or _pick(m, (512, 256, 128))
    bk = bk or _pick(kdim, (512, 384, 256, 128))
    nk = kdim // bk
    grid = (m // bm, 1, nk)
    if trans_a:
        a_spec = pl.BlockSpec((bk, bm), lambda i, j, k: (k, i))
    else:
        a_spec = pl.BlockSpec((bm, bk), lambda i, j, k: (i, k))
    b_spec = pl.BlockSpec((bk, n), lambda i, j, k: (k, 0))
    o_spec = pl.BlockSpec((bm, n), lambda i, j, k: (i, 0))
    out_dtype = out_dtype or a.dtype
    return pl.pallas_call(
        functools.partial(_mm_body, nk=nk, trans_a=trans_a),
        grid=grid,
        in_specs=[a_spec, b_spec],
        out_specs=o_spec,
        out_shape=jax.ShapeDtypeStruct((m, n), out_dtype),
        scratch_shapes=[pltpu.VMEM((bm, n), F32)],
        compiler_params=pltpu.CompilerParams(
            dimension_semantics=("parallel", "parallel", "arbitrary")),
    )(a, b)


def _cheb3(lap, rhs):
    h1 = _mm(lap, rhs)
    h2 = _mm(lap, h1)
    h3 = _mm(lap, h2)
    return h1, h2, h3


def _wstack(w, order):
    return jnp.concatenate([w[:, :, i] for i in order], axis=0).astype(CDT)


def kernel(x_0, x_1, x_2, laplacian_0, laplacian_down_1, laplacian_up_1,
           laplacian_2, b1, b2, weight_0, weight_1, weight_2):
    x0 = x_0.astype(CDT)
    x1 = x_1.astype(CDT)
    x2 = x_2.astype(CDT)
    L0 = laplacian_0.astype(CDT)
    Ld = laplacian_down_1.astype(CDT)
    Lu = laplacian_up_1.astype(CDT)
    L2 = laplacian_2.astype(CDT)
    B1 = b1.astype(CDT)
    B2 = b2.astype(CDT)

    t10 = _mm(B1, x1)
    t01 = _mm(B1, x0, trans_a=True)
    t21 = _mm(B2, x2)
    t12 = _mm(B2, x1, trans_a=True)

    h01, h02, h03 = _cheb3(L0, jnp.concatenate([x0, t10], axis=1))
    hd1, hd2, hd3 = _cheb3(Ld, jnp.concatenate([t01, x1], axis=1))
    hu1, hu2, hu3 = _cheb3(Lu, jnp.concatenate([x1, t21], axis=1))
    h21, h22, h23 = _cheb3(L2, jnp.concatenate([x2, t12], axis=1))

    X0 = jnp.concatenate([x0, t10, h01, h02, h03], axis=1)
    W0 = _wstack(weight_0, (0, 4, 1, 5, 2, 6, 3, 7))

    X1 = jnp.concatenate([t01, x1, t21, hd1, hd2, hd3, hu1, hu2, hu3], axis=1)
    W1 = _wstack(weight_1, (0, 4, 11, 1, 5, 2, 6, 3, 7, 8, 12, 9, 13, 10, 14))

    X2 = jnp.concatenate([x2, t12, h21, h22, h23], axis=1)
    W2 = _wstack(weight_2, (0, 4, 1, 5, 2, 6, 3, 7))

    y_0 = _mm(X0, W0, out_dtype=F32)
    y_1 = _mm(X1, W1, out_dtype=F32)
    y_2 = _mm(X2, W2, out_dtype=F32)
    return y_0, y_1, y_2


# speedup vs baseline: 2.2140x; 2.2140x over previous
import functools

import jax
import jax.numpy as jnp
from jax.experimental import pallas as pl
from jax.experimental.pallas import tpu as pltpu

F32 = jnp.float32
BF16 = jnp.bfloat16


def _mm_body(a_ref, b_ref, o_ref, acc_ref, *, nk, trans_a):
    k = pl.program_id(1)

    @pl.when(k == 0)
    def _():
        acc_ref[...] = jnp.zeros_like(acc_ref)

    a = a_ref[...].astype(BF16)
    b = b_ref[...].astype(BF16)
    if trans_a:
        acc_ref[...] += jax.lax.dot_general(
            a, b, (((0,), (0,)), ((), ())), preferred_element_type=F32)
    else:
        acc_ref[...] += jnp.dot(a, b, preferred_element_type=F32)

    @pl.when(k == nk - 1)
    def _():
        o_ref[...] = acc_ref[...].astype(o_ref.dtype)


def _pick(dim, cands):
    for c in cands:
        if dim % c == 0:
            return c
    return dim


def _mm(a, b, *, trans_a=False, out_dtype=BF16, bm=None, bk=None):
    if trans_a:
        kdim, m = a.shape
    else:
        m, kdim = a.shape
    _, n = b.shape
    bm = bm or _pick(m, (512, 256, 128))
    bk = bk or _pick(kdim, (512, 384, 256, 128))
    nk = kdim // bk
    grid = (m // bm, nk)
    if trans_a:
        a_spec = pl.BlockSpec((bk, bm), lambda i, k: (k, i))
    else:
        a_spec = pl.BlockSpec((bm, bk), lambda i, k: (i, k))
    b_spec = pl.BlockSpec((bk, n), lambda i, k: (k, 0))
    o_spec = pl.BlockSpec((bm, n), lambda i, k: (i, 0))
    return pl.pallas_call(
        functools.partial(_mm_body, nk=nk, trans_a=trans_a),
        grid=grid,
        in_specs=[a_spec, b_spec],
        out_specs=o_spec,
        out_shape=jax.ShapeDtypeStruct((m, n), out_dtype),
        scratch_shapes=[pltpu.VMEM((bm, n), F32)],
        compiler_params=pltpu.CompilerParams(
            dimension_semantics=("parallel", "arbitrary")),
    )(a, b)


def _cheb_body(l_ref, r_ref, o1, o2, o3, lbf, acc, *, nk, bk):
    k = pl.program_id(0)

    @pl.when(k == 0)
    def _():
        acc[...] = jnp.zeros_like(acc)

    lt = l_ref[...].astype(BF16)
    lbf[k] = lt
    acc[...] += jnp.dot(lt, r_ref[...], preferred_element_type=F32)

    @pl.when(k == nk - 1)
    def _():
        h1 = acc[...].astype(BF16)
        o1[...] = h1
        h2 = sum(jnp.dot(lbf[j], h1[j * bk:(j + 1) * bk, :],
                         preferred_element_type=F32)
                 for j in range(nk)).astype(BF16)
        o2[...] = h2
        h3 = sum(jnp.dot(lbf[j], h2[j * bk:(j + 1) * bk, :],
                         preferred_element_type=F32)
                 for j in range(nk)).astype(BF16)
        o3[...] = h3


def _cheb3(lap, rhs, *, bk=512):
    n = lap.shape[0]
    w = rhs.shape[1]
    nk = n // bk
    h_shape = jax.ShapeDtypeStruct((n, w), BF16)
    return pl.pallas_call(
        functools.partial(_cheb_body, nk=nk, bk=bk),
        grid=(nk,),
        in_specs=[pl.BlockSpec((n, bk), lambda k: (0, k)),
                  pl.BlockSpec((bk, w), lambda k: (k, 0))],
        out_specs=[pl.BlockSpec((n, w), lambda k: (0, 0))] * 3,
        out_shape=[h_shape, h_shape, h_shape],
        scratch_shapes=[pltpu.VMEM((nk, n, bk), BF16),
                        pltpu.VMEM((n, w), F32)],
        compiler_params=pltpu.CompilerParams(
            dimension_semantics=("arbitrary",)),
    )(lap, rhs)


def _proj_body(*refs):
    n_in = (len(refs) - 1) // 2
    xs, ws, o = refs[:n_in], refs[n_in:2 * n_in], refs[-1]
    acc = jnp.zeros(o.shape, F32)
    for x, w in zip(xs, ws):
        acc += jnp.dot(x[...], w[...], preferred_element_type=F32)
    o[...] = acc


def _proj(xs, ws, *, bm=512):
    m = xs[0].shape[0]
    n = ws[0].shape[1]
    in_specs = ([pl.BlockSpec((bm, x.shape[1]), lambda i: (i, 0)) for x in xs]
                + [pl.BlockSpec(w.shape, lambda i: (0, 0)) for w in ws])
    return pl.pallas_call(
        _proj_body,
        grid=(m // bm,),
        in_specs=in_specs,
        out_specs=pl.BlockSpec((bm, n), lambda i: (i, 0)),
        out_shape=jax.ShapeDtypeStruct((m, n), F32),
        compiler_params=pltpu.CompilerParams(
            dimension_semantics=("parallel",)),
    )(*xs, *ws)


def _wstack(w, order):
    return jnp.concatenate([w[:, :, i] for i in order], axis=0).astype(BF16)


def kernel(x_0, x_1, x_2, laplacian_0, laplacian_down_1, laplacian_up_1,
           laplacian_2, b1, b2, weight_0, weight_1, weight_2):
    x0 = x_0.astype(BF16)
    x1 = x_1.astype(BF16)
    x2 = x_2.astype(BF16)

    t10 = _mm(b1, x1)
    t01 = _mm(b1, x0, trans_a=True)
    t21 = _mm(b2, x2)
    t12 = _mm(b2, x1, trans_a=True)

    r0 = jnp.concatenate([x0, t10], axis=1)
    rd = jnp.concatenate([t01, x1], axis=1)
    ru = jnp.concatenate([x1, t21], axis=1)
    r2 = jnp.concatenate([x2, t12], axis=1)

    h01, h02, h03 = _cheb3(laplacian_0, r0)
    hd1, hd2, hd3 = _cheb3(laplacian_down_1, rd)
    hu1, hu2, hu3 = _cheb3(laplacian_up_1, ru)
    h21, h22, h23 = _cheb3(laplacian_2, r2)

    w0 = weight_0.astype(F32)
    W0 = [_wstack(w0, (0, 4)), _wstack(w0, (1, 5)),
          _wstack(w0, (2, 6)), _wstack(w0, (3, 7))]
    y_0 = _proj([r0, h01, h02, h03], W0)

    w1 = weight_1.astype(F32)
    W1 = [_wstack(w1, (0, 4)), _wstack(w1, (11,)),
          _wstack(w1, (1, 5)), _wstack(w1, (2, 6)), _wstack(w1, (3, 7)),
          _wstack(w1, (8, 12)), _wstack(w1, (9, 13)), _wstack(w1, (10, 14))]
    y_1 = _proj([rd, t21, hd1, hd2, hd3, hu1, hu2, hu3], W1)

    w2 = weight_2.astype(F32)
    W2 = [_wstack(w2, (0, 4)), _wstack(w2, (1, 5)),
          _wstack(w2, (2, 6)), _wstack(w2, (3, 7))]
    y_2 = _proj([r2, h21, h22, h23], W2)

    return y_0, y_1, y_2


# speedup vs baseline: 2.9132x; 1.3158x over previous
import functools

import jax
import jax.numpy as jnp
from jax.experimental import pallas as pl
from jax.experimental.pallas import tpu as pltpu

F32 = jnp.float32
BF16 = jnp.bfloat16


def _cross_body(b_ref, u_ref, v_ref, c1_ref, c2_ref, acc1, acc2, *, nm, nk, bk):
    m, k = pl.program_id(0), pl.program_id(1)

    @pl.when(k == 0)
    def _():
        acc1[...] = jnp.zeros_like(acc1)

    bt = b_ref[...].astype(BF16)
    acc1[...] += jnp.dot(bt, u_ref[...], preferred_element_type=F32)
    contrib = jax.lax.dot_general(
        bt, v_ref[...], (((0,), (0,)), ((), ())), preferred_element_type=F32)

    @pl.when(m == 0)
    def _():
        acc2[k] = contrib

    @pl.when(m > 0)
    def _():
        acc2[k] += contrib

    @pl.when(k == nk - 1)
    def _():
        c1_ref[...] = acc1[...].astype(c1_ref.dtype)

    @pl.when((m == nm - 1) & (k == nk - 1))
    def _():
        c2_ref[...] = acc2[...].reshape(c2_ref.shape).astype(c2_ref.dtype)


def _cross(b, u, v, *, bm=512, bk=512):
    m, k = b.shape
    du, dv = u.shape[1], v.shape[1]
    nm, nk = m // bm, k // bk
    return pl.pallas_call(
        functools.partial(_cross_body, nm=nm, nk=nk, bk=bk),
        grid=(nm, nk),
        in_specs=[pl.BlockSpec((bm, bk), lambda i, j: (i, j)),
                  pl.BlockSpec((bk, du), lambda i, j: (j, 0)),
                  pl.BlockSpec((bm, dv), lambda i, j: (i, 0))],
        out_specs=[pl.BlockSpec((bm, du), lambda i, j: (i, 0)),
                   pl.BlockSpec((k, dv), lambda i, j: (0, 0))],
        out_shape=[jax.ShapeDtypeStruct((m, du), BF16),
                   jax.ShapeDtypeStruct((k, dv), BF16)],
        scratch_shapes=[pltpu.VMEM((bm, du), F32),
                        pltpu.VMEM((nk, bk, dv), F32)],
        compiler_params=pltpu.CompilerParams(
            dimension_semantics=("arbitrary", "arbitrary")),
    )(b, u, v)


def _cheb_body(l_ref, r_ref, w_ref, *rest, nk, bk, has_carry):
    if has_carry:
        carry_ref, y_ref, lbf, rbf, acc = rest
    else:
        y_ref, lbf, rbf, acc = rest
    k = pl.program_id(0)

    @pl.when(k == 0)
    def _():
        acc[...] = jnp.zeros_like(acc)

    lt = l_ref[...].astype(BF16)
    lbf[k] = lt
    rt = r_ref[...]
    rbf[k] = rt
    acc[...] += jnp.dot(lt, rt, preferred_element_type=F32)

    @pl.when(k == nk - 1)
    def _():
        w = w_ref[...]
        h1 = acc[...].astype(BF16)
        h2 = sum(jnp.dot(lbf[j], h1[j * bk:(j + 1) * bk, :],
                         preferred_element_type=F32)
                 for j in range(nk)).astype(BF16)
        h3 = sum(jnp.dot(lbf[j], h2[j * bk:(j + 1) * bk, :],
                         preferred_element_type=F32)
                 for j in range(nk)).astype(BF16)
        r_full = rbf[...].reshape(h1.shape)
        y = (jnp.dot(r_full, w[0:256], preferred_element_type=F32)
             + jnp.dot(h1, w[256:512], preferred_element_type=F32)
             + jnp.dot(h2, w[512:768], preferred_element_type=F32)
             + jnp.dot(h3, w[768:1024], preferred_element_type=F32))
        if has_carry:
            y += carry_ref[...]
        y_ref[...] = y


def _cheb_proj(lap, rhs, w, carry=None, *, bk=512, out_dtype=F32):
    n = lap.shape[0]
    width = rhs.shape[1]
    nk = n // bk
    dout = w.shape[1]
    in_specs = [pl.BlockSpec((n, bk), lambda k: (0, k)),
                pl.BlockSpec((bk, width), lambda k: (k, 0)),
                pl.BlockSpec(w.shape, lambda k: (0, 0))]
    ops = [lap, rhs, w]
    if carry is not None:
        in_specs.append(pl.BlockSpec(carry.shape, lambda k: (0, 0)))
        ops.append(carry)
    return pl.pallas_call(
        functools.partial(_cheb_body, nk=nk, bk=bk, has_carry=carry is not None),
        grid=(nk,),
        in_specs=in_specs,
        out_specs=pl.BlockSpec((n, dout), lambda k: (0, 0)),
        out_shape=jax.ShapeDtypeStruct((n, dout), out_dtype),
        scratch_shapes=[pltpu.VMEM((nk, n, bk), BF16),
                        pltpu.VMEM((nk, bk, width), BF16),
                        pltpu.VMEM((n, width), F32)],
        compiler_params=pltpu.CompilerParams(
            dimension_semantics=("arbitrary",)),
    )(*ops)


def _wstack(w, pairs):
    zero = jnp.zeros(w.shape[:2], w.dtype)
    blocks = []
    for a, b in pairs:
        blocks.append(zero if a is None else w[:, :, a])
        blocks.append(zero if b is None else w[:, :, b])
    return jnp.concatenate(blocks, axis=0).astype(BF16)


def kernel(x_0, x_1, x_2, laplacian_0, laplacian_down_1, laplacian_up_1,
           laplacian_2, b1, b2, weight_0, weight_1, weight_2):
    x0 = x_0.astype(BF16)
    x1 = x_1.astype(BF16)
    x2 = x_2.astype(BF16)

    t10, t01 = _cross(b1, x1, x0)
    t21, t12 = _cross(b2, x2, x1)

    r0 = jnp.concatenate([x0, t10], axis=1)
    rd = jnp.concatenate([t01, x1], axis=1)
    ru = jnp.concatenate([x1, t21], axis=1)
    r2 = jnp.concatenate([x2, t12], axis=1)

    W0 = _wstack(weight_0, ((0, 4), (1, 5), (2, 6), (3, 7)))
    Wd = _wstack(weight_1, ((0, 4), (1, 5), (2, 6), (3, 7)))
    Wu = _wstack(weight_1, ((None, 11), (8, 12), (9, 13), (10, 14)))
    W2 = _wstack(weight_2, ((0, 4), (1, 5), (2, 6), (3, 7)))

    y_0 = _cheb_proj(laplacian_0, r0, W0)
    p_d = _cheb_proj(laplacian_down_1, rd, Wd)
    y_1 = _cheb_proj(laplacian_up_1, ru, Wu, carry=p_d)
    y_2 = _cheb_proj(laplacian_2, r2, W2)

    return y_0, y_1, y_2


# speedup vs baseline: 3.2612x; 1.1195x over previous
import functools

import jax
import jax.numpy as jnp
from jax.experimental import pallas as pl
from jax.experimental.pallas import tpu as pltpu

F32 = jnp.float32
BF16 = jnp.bfloat16


def _cross_body(b_ref, u_ref, v_ref, c1_ref, c2_ref, acc1, acc2, *, nm, nk, bk):
    m, k = pl.program_id(0), pl.program_id(1)

    @pl.when(k == 0)
    def _():
        acc1[...] = jnp.zeros_like(acc1)

    bt = b_ref[...].astype(BF16)
    acc1[...] += jnp.dot(bt, u_ref[...], preferred_element_type=F32)
    contrib = jax.lax.dot_general(
        bt, v_ref[...], (((0,), (0,)), ((), ())), preferred_element_type=F32)

    @pl.when(m == 0)
    def _():
        acc2[k] = contrib

    @pl.when(m > 0)
    def _():
        acc2[k] += contrib

    @pl.when(k == nk - 1)
    def _():
        c1_ref[...] = acc1[...].astype(c1_ref.dtype)

    @pl.when((m == nm - 1) & (k == nk - 1))
    def _():
        c2_ref[...] = acc2[...].reshape(c2_ref.shape).astype(c2_ref.dtype)


def _cross(b, u, v, *, bm=512, bk=512):
    m, k = b.shape
    du, dv = u.shape[1], v.shape[1]
    nm, nk = m // bm, k // bk
    return pl.pallas_call(
        functools.partial(_cross_body, nm=nm, nk=nk, bk=bk),
        grid=(nm, nk),
        in_specs=[pl.BlockSpec((bm, bk), lambda i, j: (i, j)),
                  pl.BlockSpec((bk, du), lambda i, j: (j, 0)),
                  pl.BlockSpec((bm, dv), lambda i, j: (i, 0))],
        out_specs=[pl.BlockSpec((bm, du), lambda i, j: (i, 0)),
                   pl.BlockSpec((k, dv), lambda i, j: (0, 0))],
        out_shape=[jax.ShapeDtypeStruct((m, du), BF16),
                   jax.ShapeDtypeStruct((k, dv), BF16)],
        scratch_shapes=[pltpu.VMEM((bm, du), F32),
                        pltpu.VMEM((nk, bk, dv), F32)],
        compiler_params=pltpu.CompilerParams(
            dimension_semantics=("arbitrary", "arbitrary")),
    )(b, u, v)


def _proj4(r, h1, h2, h3, w, carry=None):
    y = (jnp.dot(r, w[0:256], preferred_element_type=F32)
         + jnp.dot(h1, w[256:512], preferred_element_type=F32)
         + jnp.dot(h2, w[512:768], preferred_element_type=F32)
         + jnp.dot(h3, w[768:1024], preferred_element_type=F32))
    if carry is not None:
        y += carry
    return y


def _hop(lbf, h, n, *, cb=512, chunks=None):
    rng = range(n // cb) if chunks is None else chunks
    return sum(jnp.dot(lbf[:, c * cb:(c + 1) * cb], h[c * cb:(c + 1) * cb, :],
                       preferred_element_type=F32) for c in rng)


def _hop_to(out_ref, lbf, h_ref, n, *, mb=1024, cb=512):
    for m0 in range(0, n, mb):
        part = sum(jnp.dot(lbf[m0:m0 + mb, c * cb:(c + 1) * cb],
                           h_ref[c * cb:(c + 1) * cb, :],
                           preferred_element_type=F32)
                   for c in range(n // cb))
        out_ref[m0:m0 + mb, :] = part.astype(BF16)


def _tail_to(y_ref, lbf, r_ref, h1_ref, h2_ref, w_ref, carry_ref, n,
             *, mb=1024, cb=512):
    w = w_ref[...]
    for m0 in range(0, n, mb):
        h3p = sum(jnp.dot(lbf[m0:m0 + mb, c * cb:(c + 1) * cb],
                          h2_ref[c * cb:(c + 1) * cb, :],
                          preferred_element_type=F32)
                  for c in range(n // cb)).astype(BF16)
        y = (jnp.dot(r_ref[m0:m0 + mb, :], w[0:256],
                     preferred_element_type=F32)
             + jnp.dot(h1_ref[m0:m0 + mb, :], w[256:512],
                       preferred_element_type=F32)
             + jnp.dot(h2_ref[m0:m0 + mb, :], w[512:768],
                       preferred_element_type=F32)
             + jnp.dot(h3p, w[768:1024], preferred_element_type=F32))
        if carry_ref is not None:
            y += carry_ref[m0:m0 + mb, :]
        y_ref[m0:m0 + mb, :] = y


def _k1_body(l0_ref, ld_ref, r0_ref, rd_ref, w0_ref, wd_ref,
             y0_ref, pd_ref, lbf0, lbfd, acc0, accd, h1b, h2b,
             *, n0k, ndk, bk):
    s = pl.program_id(0)

    @pl.when(s == 0)
    def _():
        acc0[...] = jnp.zeros_like(acc0)
        accd[...] = jnp.zeros_like(accd)

    @pl.when(s < n0k)
    def _():
        lt = l0_ref[...].astype(BF16)
        lbf0[:, pl.ds(s * bk, bk)] = lt
        acc0[...] += jnp.dot(lt, r0_ref[pl.ds(s * bk, bk), :],
                             preferred_element_type=F32)

    @pl.when((s >= n0k) & (s < n0k + ndk))
    def _():
        k = s - n0k
        lt = ld_ref[...].astype(BF16)
        lbfd[:, pl.ds(k * bk, bk)] = lt
        accd[...] += jnp.dot(lt, rd_ref[pl.ds(k * bk, bk), :],
                             preferred_element_type=F32)

    @pl.when(s == n0k)
    def _():
        h1b[0:1024, :] = acc0[...].astype(BF16)
        _hop_to(h2b, lbf0, h1b, 1024)
        _tail_to(y0_ref, lbf0, r0_ref, h1b, h2b, w0_ref, None, 1024)

    @pl.when(s == n0k + ndk - 1)
    def _():
        h1b[...] = accd[...].astype(BF16)
        _hop_to(h2b, lbfd, h1b, 3072)
        _tail_to(pd_ref, lbfd, rd_ref, h1b, h2b, wd_ref, None, 3072)


def _k1(l0, ld, r0, rd, w0, wd, *, bk=512):
    n0, nd = l0.shape[0], ld.shape[0]
    n0k, ndk = n0 // bk, nd // bk
    c0 = n0k - 1
    cd = ndk - 1
    return pl.pallas_call(
        functools.partial(_k1_body, n0k=n0k, ndk=ndk, bk=bk),
        grid=(n0k + ndk,),
        in_specs=[
            pl.BlockSpec((n0, bk), lambda s: (0, jnp.clip(s, 0, c0))),
            pl.BlockSpec((nd, bk), lambda s: (0, jnp.clip(s - c0 - 1, 0, cd))),
            pl.BlockSpec((n0, 256), lambda s: (0, 0)),
            pl.BlockSpec((nd, 256), lambda s: (0, 0)),
            pl.BlockSpec((1024, 128), lambda s: (0, 0)),
            pl.BlockSpec((1024, 128), lambda s: (0, 0)),
        ],
        out_specs=[pl.BlockSpec((n0, 128), lambda s: (0, 0)),
                   pl.BlockSpec((nd, 128), lambda s: (0, 0))],
        out_shape=[jax.ShapeDtypeStruct((n0, 128), F32),
                   jax.ShapeDtypeStruct((nd, 128), F32)],
        scratch_shapes=[pltpu.VMEM((n0, n0), BF16),
                        pltpu.VMEM((nd, nd), BF16),
                        pltpu.VMEM((n0, 256), F32),
                        pltpu.VMEM((nd, 256), F32),
                        pltpu.VMEM((nd, 256), BF16),
                        pltpu.VMEM((nd, 256), BF16)],
        compiler_params=pltpu.CompilerParams(
            dimension_semantics=("arbitrary",)),
    )(l0, ld, r0, rd, w0, wd)


def _k2_body(lu_ref, l2_ref, ru_ref, r2_ref, wu_ref, w2_ref, pd_ref,
             y1_ref, y2_ref, lbfu, lbf2, accu, acc2, h1u, h2u, *, nuk, n2k, bk):
    s = pl.program_id(0)
    last = nuk + n2k

    @pl.when(s == 0)
    def _():
        accu[...] = jnp.zeros_like(accu)
        acc2[...] = jnp.zeros_like(acc2)

    @pl.when(s < nuk)
    def _():
        lt = lu_ref[...].astype(BF16)
        lbfu[:, pl.ds(s * bk, bk)] = lt
        accu[...] += jnp.dot(lt, ru_ref[pl.ds(s * bk, bk), :],
                             preferred_element_type=F32)

    @pl.when((s >= nuk) & (s < nuk + n2k))
    def _():
        k = s - nuk
        lt = l2_ref[...].astype(BF16)
        lbf2[:, pl.ds(k * bk, bk)] = lt
        acc2[...] += jnp.dot(lt, r2_ref[pl.ds(k * bk, bk), :],
                             preferred_element_type=F32)

    @pl.when(s == nuk)
    def _():
        h1u[...] = accu[...].astype(BF16)
        accu[...] = _hop(lbfu, h1u, 3072, chunks=[0])

    for c in range(1, 6):
        @pl.when(s == nuk + c)
        def _(c=c):
            accu[...] += _hop(lbfu, h1u, 3072, chunks=[c])

    @pl.when(s == nuk + 6)
    def _():
        h2u[...] = accu[...].astype(BF16)
        accu[...] = _hop(lbfu, h2u, 3072, chunks=[0])

    @pl.when(s == nuk + 7)
    def _():
        accu[...] += _hop(lbfu, h2u, 3072, chunks=[1])

    @pl.when(s == last)
    def _():
        wu = wu_ref[...]
        for m0 in range(0, 3072, 1024):
            h3p = (accu[m0:m0 + 1024, :]
                   + sum(jnp.dot(lbfu[m0:m0 + 1024, c * 512:(c + 1) * 512],
                                 h2u[c * 512:(c + 1) * 512, :],
                                 preferred_element_type=F32)
                         for c in (2, 3, 4, 5))).astype(BF16)
            y1_ref[m0:m0 + 1024, :] = (
                jnp.dot(ru_ref[m0:m0 + 1024, :], wu[0:256],
                        preferred_element_type=F32)
                + jnp.dot(h1u[m0:m0 + 1024, :], wu[256:512],
                          preferred_element_type=F32)
                + jnp.dot(h2u[m0:m0 + 1024, :], wu[512:768],
                          preferred_element_type=F32)
                + jnp.dot(h3p, wu[768:1024], preferred_element_type=F32)
                + pd_ref[m0:m0 + 1024, :])
        h1u[0:2048, :] = acc2[...].astype(BF16)
        _hop_to(h2u, lbf2, h1u, 2048)
        _tail_to(y2_ref, lbf2, r2_ref, h1u, h2u, w2_ref, None, 2048)


def _k2(lu, l2, ru, r2, wu, w2, pd, *, bk=256):
    nu, n2 = lu.shape[0], l2.shape[0]
    nuk, n2k = nu // bk, n2 // bk
    cu = nuk - 1
    c2 = n2k - 1
    return pl.pallas_call(
        functools.partial(_k2_body, nuk=nuk, n2k=n2k, bk=bk),
        grid=(nuk + n2k + 1,),
        in_specs=[
            pl.BlockSpec((nu, bk), lambda s: (0, jnp.clip(s, 0, cu))),
            pl.BlockSpec((n2, bk), lambda s: (0, jnp.clip(s - cu - 1, 0, c2))),
            pl.BlockSpec((nu, 256), lambda s: (0, 0)),
            pl.BlockSpec((n2, 256), lambda s: (0, 0)),
            pl.BlockSpec((1024, 128), lambda s: (0, 0)),
            pl.BlockSpec((1024, 128), lambda s: (0, 0)),
            pl.BlockSpec((nu, 128), lambda s: (0, 0)),
        ],
        out_specs=[pl.BlockSpec((nu, 128), lambda s: (0, 0)),
                   pl.BlockSpec((n2, 128), lambda s: (0, 0))],
        out_shape=[jax.ShapeDtypeStruct((nu, 128), F32),
                   jax.ShapeDtypeStruct((n2, 128), F32)],
        scratch_shapes=[pltpu.VMEM((nu, nu), BF16),
                        pltpu.VMEM((n2, n2), BF16),
                        pltpu.VMEM((nu, 256), F32),
                        pltpu.VMEM((n2, 256), F32),
                        pltpu.VMEM((nu, 256), BF16),
                        pltpu.VMEM((nu, 256), BF16)],
        compiler_params=pltpu.CompilerParams(
            dimension_semantics=("arbitrary",)),
    )(lu, l2, ru, r2, wu, w2, pd)


def _wstack(w, pairs):
    zero = jnp.zeros(w.shape[:2], w.dtype)
    blocks = []
    for a, b in pairs:
        blocks.append(zero if a is None else w[:, :, a])
        blocks.append(zero if b is None else w[:, :, b])
    return jnp.concatenate(blocks, axis=0).astype(BF16)


def kernel(x_0, x_1, x_2, laplacian_0, laplacian_down_1, laplacian_up_1,
           laplacian_2, b1, b2, weight_0, weight_1, weight_2):
    x0 = x_0.astype(BF16)
    x1 = x_1.astype(BF16)
    x2 = x_2.astype(BF16)

    t10, t01 = _cross(b1, x1, x0)
    t21, t12 = _cross(b2, x2, x1)

    r0 = jnp.concatenate([x0, t10], axis=1)
    rd = jnp.concatenate([t01, x1], axis=1)
    ru = jnp.concatenate([x1, t21], axis=1)
    r2 = jnp.concatenate([x2, t12], axis=1)

    W0 = _wstack(weight_0, ((0, 4), (1, 5), (2, 6), (3, 7)))
    Wd = _wstack(weight_1, ((0, 4), (1, 5), (2, 6), (3, 7)))
    Wu = _wstack(weight_1, ((None, 11), (8, 12), (9, 13), (10, 14)))
    W2 = _wstack(weight_2, ((0, 4), (1, 5), (2, 6), (3, 7)))

    y_0, p_d = _k1(laplacian_0, laplacian_down_1, r0, rd, W0, Wd)
    y_1, y_2 = _k2(laplacian_up_1, laplacian_2, ru, r2, Wu, W2, p_d)

    return y_0, y_1, y_2


# speedup vs baseline: 3.6282x; 1.1125x over previous
import functools

import jax
import jax.numpy as jnp
from jax.experimental import pallas as pl
from jax.experimental.pallas import tpu as pltpu

F32 = jnp.float32
BF16 = jnp.bfloat16


def _cross_body(b_ref, u_ref, v_ref, c1_ref, c2_ref, acc1, acc2, *, nm, nk, bk):
    m, k = pl.program_id(0), pl.program_id(1)

    @pl.when(k == 0)
    def _():
        acc1[...] = jnp.zeros_like(acc1)

    bt = b_ref[...].astype(BF16)
    acc1[...] += jnp.dot(bt, u_ref[...], preferred_element_type=F32)
    contrib = jax.lax.dot_general(
        bt, v_ref[...], (((0,), (0,)), ((), ())), preferred_element_type=F32)

    @pl.when(m == 0)
    def _():
        acc2[k] = contrib

    @pl.when(m > 0)
    def _():
        acc2[k] += contrib

    @pl.when(k == nk - 1)
    def _():
        c1_ref[...] = acc1[...].astype(c1_ref.dtype)

    @pl.when((m == nm - 1) & (k == nk - 1))
    def _():
        c2_ref[...] = acc2[...].reshape(c2_ref.shape).astype(c2_ref.dtype)


def _cross(b, u, v, *, bm=1024, bk=1024):
    m, k = b.shape
    du, dv = u.shape[1], v.shape[1]
    nm, nk = m // bm, k // bk
    return pl.pallas_call(
        functools.partial(_cross_body, nm=nm, nk=nk, bk=bk),
        grid=(nm, nk),
        in_specs=[pl.BlockSpec((bm, bk), lambda i, j: (i, j)),
                  pl.BlockSpec((bk, du), lambda i, j: (j, 0)),
                  pl.BlockSpec((bm, dv), lambda i, j: (i, 0))],
        out_specs=[pl.BlockSpec((bm, du), lambda i, j: (i, 0)),
                   pl.BlockSpec((k, dv), lambda i, j: (0, 0))],
        out_shape=[jax.ShapeDtypeStruct((m, du), BF16),
                   jax.ShapeDtypeStruct((k, dv), BF16)],
        scratch_shapes=[pltpu.VMEM((bm, du), F32),
                        pltpu.VMEM((nk, bk, dv), F32)],
        compiler_params=pltpu.CompilerParams(
            dimension_semantics=("arbitrary", "arbitrary")),
    )(b, u, v)


def _proj4(r, h1, h2, h3, w, carry=None):
    y = (jnp.dot(r, w[0:256], preferred_element_type=F32)
         + jnp.dot(h1, w[256:512], preferred_element_type=F32)
         + jnp.dot(h2, w[512:768], preferred_element_type=F32)
         + jnp.dot(h3, w[768:1024], preferred_element_type=F32))
    if carry is not None:
        y += carry
    return y


def _hop(lbf, h, n, *, cb=512, chunks=None):
    rng = range(n // cb) if chunks is None else chunks
    return sum(jnp.dot(lbf[:, c * cb:(c + 1) * cb], h[c * cb:(c + 1) * cb, :],
                       preferred_element_type=F32) for c in rng)


def _hop_to(out_ref, lbf, h_ref, n, *, mb=1024, cb=512):
    for m0 in range(0, n, mb):
        part = sum(jnp.dot(lbf[m0:m0 + mb, c * cb:(c + 1) * cb],
                           h_ref[c * cb:(c + 1) * cb, :],
                           preferred_element_type=F32)
                   for c in range(n // cb))
        out_ref[m0:m0 + mb, :] = part.astype(BF16)


def _tail_to(y_ref, lbf, r_ref, h1_ref, h2_ref, w_ref, carry_ref, n,
             *, mb=1024, cb=512):
    w = w_ref[...]
    for m0 in range(0, n, mb):
        h3p = sum(jnp.dot(lbf[m0:m0 + mb, c * cb:(c + 1) * cb],
                          h2_ref[c * cb:(c + 1) * cb, :],
                          preferred_element_type=F32)
                  for c in range(n // cb)).astype(BF16)
        y = (jnp.dot(r_ref[m0:m0 + mb, :], w[0:256],
                     preferred_element_type=F32)
             + jnp.dot(h1_ref[m0:m0 + mb, :], w[256:512],
                       preferred_element_type=F32)
             + jnp.dot(h2_ref[m0:m0 + mb, :], w[512:768],
                       preferred_element_type=F32)
             + jnp.dot(h3p, w[768:1024], preferred_element_type=F32))
        if carry_ref is not None:
            y += carry_ref[m0:m0 + mb, :]
        y_ref[m0:m0 + mb, :] = y


def _k1_body(l0_ref, ld_ref, r0_ref, rd_ref, w0_ref, wd_ref,
             y0_ref, pd_ref, lbf0, lbfd, acc0, accd, h1b, h2b,
             *, n0k, ndk, bk):
    s = pl.program_id(0)

    @pl.when(s == 0)
    def _():
        acc0[...] = jnp.zeros_like(acc0)
        accd[...] = jnp.zeros_like(accd)

    @pl.when(s < n0k)
    def _():
        lt = l0_ref[...].astype(BF16)
        lbf0[:, pl.ds(s * bk, bk)] = lt
        acc0[...] += jnp.dot(lt, r0_ref[pl.ds(s * bk, bk), :],
                             preferred_element_type=F32)

    @pl.when((s >= n0k) & (s < n0k + ndk))
    def _():
        k = s - n0k
        lt = ld_ref[...].astype(BF16)
        lbfd[:, pl.ds(k * bk, bk)] = lt
        accd[...] += jnp.dot(lt, rd_ref[pl.ds(k * bk, bk), :],
                             preferred_element_type=F32)

    @pl.when(s == n0k)
    def _():
        h1b[0:1024, :] = acc0[...].astype(BF16)
        _hop_to(h2b, lbf0, h1b, 1024)
        _tail_to(y0_ref, lbf0, r0_ref, h1b, h2b, w0_ref, None, 1024)

    @pl.when(s == n0k + ndk - 1)
    def _():
        h1b[...] = accd[...].astype(BF16)
        _hop_to(h2b, lbfd, h1b, 3072)
        _tail_to(pd_ref, lbfd, rd_ref, h1b, h2b, wd_ref, None, 3072)


def _k1(l0, ld, r0, rd, w0, wd, *, bk=512):
    n0, nd = l0.shape[0], ld.shape[0]
    n0k, ndk = n0 // bk, nd // bk
    c0 = n0k - 1
    cd = ndk - 1
    return pl.pallas_call(
        functools.partial(_k1_body, n0k=n0k, ndk=ndk, bk=bk),
        grid=(n0k + ndk,),
        in_specs=[
            pl.BlockSpec((n0, bk), lambda s: (0, jnp.clip(s, 0, c0))),
            pl.BlockSpec((nd, bk), lambda s: (0, jnp.clip(s - c0 - 1, 0, cd))),
            pl.BlockSpec((n0, 256), lambda s: (0, 0)),
            pl.BlockSpec((nd, 256), lambda s: (0, 0)),
            pl.BlockSpec((1024, 128), lambda s: (0, 0)),
            pl.BlockSpec((1024, 128), lambda s: (0, 0)),
        ],
        out_specs=[pl.BlockSpec((n0, 128), lambda s: (0, 0)),
                   pl.BlockSpec((nd, 128), lambda s: (0, 0))],
        out_shape=[jax.ShapeDtypeStruct((n0, 128), F32),
                   jax.ShapeDtypeStruct((nd, 128), F32)],
        scratch_shapes=[pltpu.VMEM((n0, n0), BF16),
                        pltpu.VMEM((nd, nd), BF16),
                        pltpu.VMEM((n0, 256), F32),
                        pltpu.VMEM((nd, 256), F32),
                        pltpu.VMEM((nd, 256), BF16),
                        pltpu.VMEM((nd, 256), BF16)],
        compiler_params=pltpu.CompilerParams(
            dimension_semantics=("arbitrary",)),
    )(l0, ld, r0, rd, w0, wd)


def _k2_body(lu_ref, l2_ref, ru_ref, r2_ref, wu_ref, w2_ref, pd_ref,
             y1_ref, y2_ref, lbfu, lbf2, accu, acc2, h1u, h2u,
             *, nuk, n2k, bku, bk2):
    s = pl.program_id(0)
    last = nuk + n2k

    @pl.when(s == 0)
    def _():
        accu[...] = jnp.zeros_like(accu)
        acc2[...] = jnp.zeros_like(acc2)

    @pl.when(s < nuk)
    def _():
        lt = lu_ref[...].astype(BF16)
        lbfu[:, pl.ds(s * bku, bku)] = lt
        accu[...] += jnp.dot(lt, ru_ref[pl.ds(s * bku, bku), :],
                             preferred_element_type=F32)

    @pl.when((s >= nuk) & (s < nuk + n2k))
    def _():
        k = s - nuk
        lt = l2_ref[...].astype(BF16)
        lbf2[:, pl.ds(k * bk2, bk2)] = lt
        acc2[...] += jnp.dot(lt, r2_ref[pl.ds(k * bk2, bk2), :],
                             preferred_element_type=F32)

    @pl.when(s == nuk)
    def _():
        h1u[...] = accu[...].astype(BF16)
        accu[...] = _hop(lbfu, h1u, 3072, chunks=[0])

    for c in range(1, 6):
        @pl.when(s == nuk + c)
        def _(c=c):
            accu[...] += _hop(lbfu, h1u, 3072, chunks=[c])

    @pl.when(s == nuk + 6)
    def _():
        h2u[...] = accu[...].astype(BF16)
        accu[...] = _hop(lbfu, h2u, 3072, chunks=[0])

    for c in range(1, 6):
        @pl.when(s == nuk + 6 + c)
        def _(c=c):
            accu[...] += _hop(lbfu, h2u, 3072, chunks=[c])

    @pl.when(s == nuk + 12)
    def _():
        wu = wu_ref[...]
        for m0 in range(0, 3072, 1024):
            h3p = accu[m0:m0 + 1024, :].astype(BF16)
            y1_ref[m0:m0 + 1024, :] = (
                jnp.dot(ru_ref[m0:m0 + 1024, :], wu[0:256],
                        preferred_element_type=F32)
                + jnp.dot(h1u[m0:m0 + 1024, :], wu[256:512],
                          preferred_element_type=F32)
                + jnp.dot(h2u[m0:m0 + 1024, :], wu[512:768],
                          preferred_element_type=F32)
                + jnp.dot(h3p, wu[768:1024], preferred_element_type=F32)
                + pd_ref[m0:m0 + 1024, :])

    @pl.when(s == last)
    def _():
        h1u[0:2048, :] = acc2[...].astype(BF16)
        _hop_to(h2u, lbf2, h1u, 2048)
        _tail_to(y2_ref, lbf2, r2_ref, h1u, h2u, w2_ref, None, 2048)


def _k2(lu, l2, ru, r2, wu, w2, pd, *, bku=512, bk2=128):
    nu, n2 = lu.shape[0], l2.shape[0]
    nuk, n2k = nu // bku, n2 // bk2
    cu = nuk - 1
    c2 = n2k - 1
    return pl.pallas_call(
        functools.partial(_k2_body, nuk=nuk, n2k=n2k, bku=bku, bk2=bk2),
        grid=(nuk + n2k + 1,),
        in_specs=[
            pl.BlockSpec((nu, bku), lambda s: (0, jnp.clip(s, 0, cu))),
            pl.BlockSpec((n2, bk2), lambda s: (0, jnp.clip(s - cu - 1, 0, c2))),
            pl.BlockSpec((nu, 256), lambda s: (0, 0)),
            pl.BlockSpec((n2, 256), lambda s: (0, 0)),
            pl.BlockSpec((1024, 128), lambda s: (0, 0)),
            pl.BlockSpec((1024, 128), lambda s: (0, 0)),
            pl.BlockSpec((nu, 128), lambda s: (0, 0)),
        ],
        out_specs=[pl.BlockSpec((nu, 128), lambda s: (0, 0)),
                   pl.BlockSpec((n2, 128), lambda s: (0, 0))],
        out_shape=[jax.ShapeDtypeStruct((nu, 128), F32),
                   jax.ShapeDtypeStruct((n2, 128), F32)],
        scratch_shapes=[pltpu.VMEM((nu, nu), BF16),
                        pltpu.VMEM((n2, n2), BF16),
                        pltpu.VMEM((nu, 256), F32),
                        pltpu.VMEM((n2, 256), F32),
                        pltpu.VMEM((nu, 256), BF16),
                        pltpu.VMEM((nu, 256), BF16)],
        compiler_params=pltpu.CompilerParams(
            dimension_semantics=("arbitrary",)),
    )(lu, l2, ru, r2, wu, w2, pd)


def _wstack(w, pairs):
    zero = jnp.zeros(w.shape[:2], w.dtype)
    blocks = []
    for a, b in pairs:
        blocks.append(zero if a is None else w[:, :, a])
        blocks.append(zero if b is None else w[:, :, b])
    return jnp.concatenate(blocks, axis=0).astype(BF16)


def kernel(x_0, x_1, x_2, laplacian_0, laplacian_down_1, laplacian_up_1,
           laplacian_2, b1, b2, weight_0, weight_1, weight_2):
    x0 = x_0.astype(BF16)
    x1 = x_1.astype(BF16)
    x2 = x_2.astype(BF16)

    t10, t01 = _cross(b1, x1, x0)
    t21, t12 = _cross(b2, x2, x1)

    r0 = jnp.concatenate([x0, t10], axis=1)
    rd = jnp.concatenate([t01, x1], axis=1)
    ru = jnp.concatenate([x1, t21], axis=1)
    r2 = jnp.concatenate([x2, t12], axis=1)

    W0 = _wstack(weight_0, ((0, 4), (1, 5), (2, 6), (3, 7)))
    Wd = _wstack(weight_1, ((0, 4), (1, 5), (2, 6), (3, 7)))
    Wu = _wstack(weight_1, ((None, 11), (8, 12), (9, 13), (10, 14)))
    W2 = _wstack(weight_2, ((0, 4), (1, 5), (2, 6), (3, 7)))

    y_0, p_d = _k1(laplacian_0, laplacian_down_1, r0, rd, W0, Wd)
    y_1, y_2 = _k2(laplacian_up_1, laplacian_2, ru, r2, Wu, W2, p_d)

    return y_0, y_1, y_2
